# baseline (speedup 1.0000x reference)
import functools

import jax
import jax.numpy as jnp
import numpy as np
from jax import lax
from jax.experimental import pallas as pl
from jax.experimental.pallas import tpu as pltpu
from jax.experimental.pallas import tpu_sc as plsc

MUL = 64
NB = 8
RMAX = 5.0
EP = 163840
GATHER_CHUNKS = 40
SCATTER_CHUNKS = 80


def _up_body(nf_ref, wu0_ref, wu1_ref, out_ref):
    nf = nf_ref[...]
    wu0 = wu0_ref[...]
    wu1 = wu1_ref[...]
    parts = [jnp.dot(nf[:, :MUL], wu0, preferred_element_type=jnp.float32)]
    for i in range(3):
        parts.append(jnp.dot(nf[:, MUL + 64 * i:MUL + 64 * (i + 1)], wu1,
                             preferred_element_type=jnp.float32))
    out_ref[...] = (jnp.concatenate(parts, axis=1)
                    * (1.0 / np.sqrt(MUL))).astype(jnp.bfloat16)


def _linear_up(node_feats, W_up0, W_up1):
    n = node_feats.shape[0]
    bn = 2000
    grid = n // bn
    return pl.pallas_call(
        _up_body,
        grid=(grid,),
        in_specs=[
            pl.BlockSpec((bn, 4 * MUL), lambda i: (i, 0)),
            pl.BlockSpec((MUL, MUL), lambda i: (0, 0)),
            pl.BlockSpec((MUL, MUL), lambda i: (0, 0)),
        ],
        out_specs=pl.BlockSpec((bn, 4 * MUL), lambda i: (i, 0)),
        out_shape=jax.ShapeDtypeStruct((n, 4 * MUL), jnp.bfloat16),
    )(node_feats, W_up0, W_up1)


def _edge_body(ev_ref, el_ref, xs_ref, w1_ref, b1_ref, w2_ref, b2_ref,
               w3_ref, b3_ref, wl0_ref, wl1_ref, scale_ref, outa_ref, outb_ref,
               *, block_e, e_valid):
    ev = ev_ref[...]
    n2 = jnp.sum(ev * ev, axis=1, keepdims=True)
    u = ev / jnp.clip(jnp.sqrt(n2), 1e-12, None)
    sh = np.sqrt(3.0).astype(np.float32) * u

    r = el_ref[...]
    x = jnp.clip(r * (1.0 / RMAX), 0.0, 1.0)
    centers = (lax.broadcasted_iota(jnp.int32, (1, NB), 1).astype(jnp.float32)
               * (1.0 / (NB - 1)))
    d = (x - centers) * float(NB - 1)
    emb = jnp.exp(-0.5 * d * d) * np.sqrt(float(NB)).astype(np.float32)
    emb = emb * (r <= RMAX).astype(jnp.float32)

    h = emb @ w1_ref[...] + b1_ref[...]
    h = h * jax.nn.sigmoid(h)
    h = h @ w2_ref[...] + b2_ref[...]
    h = h * jax.nn.sigmoid(h)
    w = h @ w3_ref[...] + b3_ref[...]

    xs = xs_ref[...].astype(jnp.float32)
    x0s = xs[:, :MUL]
    x1s = [xs[:, MUL + 64 * i:MUL + 64 * (i + 1)] for i in range(3)]
    dot31 = (x1s[0] * sh[:, 0:1] + x1s[1] * sh[:, 1:2] + x1s[2] * sh[:, 2:3])

    w0 = w[:, 0 * MUL:1 * MUL]
    w1 = w[:, 1 * MUL:2 * MUL]
    w2 = w[:, 2 * MUL:3 * MUL]
    w3 = w[:, 3 * MUL:4 * MUL]

    wl0 = wl0_ref[...]
    wl1 = wl1_ref[...]
    mid0 = jnp.concatenate([w0 * x0s, w3 * dot31 * (1.0 / np.sqrt(3.0))], axis=1)
    ys = [jnp.dot(mid0, wl0, preferred_element_type=jnp.float32)]
    w1x0 = w1 * x0s
    for i in range(3):
        mid1 = jnp.concatenate([w1x0 * sh[:, i:i + 1], w2 * x1s[i]], axis=1)
        ys.append(jnp.dot(mid1, wl1, preferred_element_type=jnp.float32))
    sc = scale_ref[0, 0]
    row = (pl.program_id(0) * block_e +
           lax.broadcasted_iota(jnp.int32, (ys[0].shape[0], 2 * MUL), 0))
    mask = (row < e_valid).astype(jnp.float32) * sc
    outa_ref[...] = jnp.concatenate(ys[:2], axis=1) * mask
    outb_ref[...] = jnp.concatenate(ys[2:], axis=1) * mask


def _edge_compute(ev_p, el_p, xs, W1, b1, W2, b2, W3, b3, W_l0, W_l1, scale,
                  e_valid):
    be = 2048
    grid = EP // be
    body = functools.partial(_edge_body, block_e=be, e_valid=e_valid)
    return pl.pallas_call(
        body,
        grid=(grid,),
        in_specs=[
            pl.BlockSpec((be, 3), lambda i: (i, 0)),
            pl.BlockSpec((be, 1), lambda i: (i, 0)),
            pl.BlockSpec((be, 4 * MUL), lambda i: (i, 0)),
            pl.BlockSpec((NB, 64), lambda i: (0, 0)),
            pl.BlockSpec((1, 64), lambda i: (0, 0)),
            pl.BlockSpec((64, 64), lambda i: (0, 0)),
            pl.BlockSpec((1, 64), lambda i: (0, 0)),
            pl.BlockSpec((64, 4 * MUL), lambda i: (0, 0)),
            pl.BlockSpec((1, 4 * MUL), lambda i: (0, 0)),
            pl.BlockSpec((2 * MUL, MUL), lambda i: (0, 0)),
            pl.BlockSpec((2 * MUL, MUL), lambda i: (0, 0)),
            pl.BlockSpec((1, 1), lambda i: (0, 0)),
        ],
        out_specs=[pl.BlockSpec((be, 2 * MUL), lambda i: (i, 0)),
                   pl.BlockSpec((be, 2 * MUL), lambda i: (i, 0))],
        out_shape=[jax.ShapeDtypeStruct((EP, 2 * MUL), jnp.float32),
                   jax.ShapeDtypeStruct((EP, 2 * MUL), jnp.float32)],
    )(ev_p, el_p, xs, W1, b1[None, :], W2, b2[None, :], W3, b3[None, :],
      W_l0, W_l1, scale)


def _sc_gather(x_up, senders3):
    mesh = plsc.VectorSubcoreMesh(core_axis_name="c", subcore_axis_name="s")
    d = 4 * MUL

    @functools.partial(
        pl.kernel, mesh=mesh,
        out_type=jax.ShapeDtypeStruct((EP, d // 2), jnp.int32),
        scratch_types=[
            pltpu.VMEM((GATHER_CHUNKS, 128), jnp.int32),
            pltpu.VMEM((128, d // 2), jnp.int32),
            pltpu.VMEM((128, d // 2), jnp.int32),
            pltpu.SemaphoreType.DMA,
            pltpu.SemaphoreType.DMA,
        ],
    )
    def gk(table_hbm, idx_hbm, out_hbm, idx_v, rows0_v, rows1_v, sem0, sem1):
        c = lax.axis_index("c")
        s = lax.axis_index("s")
        wid = s * 2 + c
        pltpu.sync_copy(idx_hbm.at[wid], idx_v)
        base = wid * (GATHER_CHUNKS * 128)

        def body(t, carry):
            j0 = 2 * t
            cp0 = pltpu.async_copy(table_hbm.at[idx_v.at[j0]], rows0_v, sem0)
            cp1 = pltpu.async_copy(table_hbm.at[idx_v.at[j0 + 1]], rows1_v, sem1)
            cp0.wait()
            pltpu.sync_copy(rows0_v, out_hbm.at[pl.ds(base + j0 * 128, 128)])
            cp1.wait()
            pltpu.sync_copy(rows1_v, out_hbm.at[pl.ds(base + j0 * 128 + 128, 128)])
            return carry

        lax.fori_loop(0, GATHER_CHUNKS // 2, body, 0)

    return gk(x_up, senders3)


def _sc_scatter(ya, yb, recv3, zeros_half, n_pad):
    mesh = plsc.VectorSubcoreMesh(core_axis_name="c", subcore_axis_name="s")
    rows_per_tile = n_pad // 16

    @functools.partial(
        pl.kernel, mesh=mesh,
        out_type=jax.ShapeDtypeStruct((2, n_pad, 128), jnp.float32),
        scratch_types=[
            pltpu.VMEM_SHARED((n_pad, 128), jnp.float32),
            pltpu.VMEM((SCATTER_CHUNKS, 128), jnp.int32),
            pltpu.VMEM((128, 128), jnp.float32),
        ],
    )
    def sk(ya_hbm, yb_hbm, idx_hbm, zeros_hbm, out_hbm, accum, idx_v, rows_v):
        c = lax.axis_index("c")
        s = lax.axis_index("s")
        nslice = pl.ds(s * rows_per_tile, rows_per_tile)
        pltpu.sync_copy(zeros_hbm.at[nslice], accum.at[nslice])
        pltpu.sync_copy(idx_hbm.at[s], idx_v)
        plsc.subcore_barrier()
        base = s * (SCATTER_CHUNKS * 128)

        def body(j, carry):
            sl = pl.ds(base + j * 128, 128)

            @pl.when(c == 0)
            def _():
                pltpu.sync_copy(ya_hbm.at[sl], rows_v)

            @pl.when(c == 1)
            def _():
                pltpu.sync_copy(yb_hbm.at[sl], rows_v)

            pltpu.sync_copy(rows_v, accum.at[idx_v.at[j]], add=True)
            return carry

        lax.fori_loop(0, SCATTER_CHUNKS, body, 0)
        plsc.subcore_barrier()
        pltpu.sync_copy(accum.at[nslice], out_hbm.at[c].at[nslice])

    return sk(ya, yb, recv3, zeros_half)


def kernel(node_feats, edge_vec, edge_length, senders, receivers, num_nodes,
           W1, b1, W2, b2, W3, b3, W_up0, W_up1, W_l0, W_l1):
    n = node_feats.shape[0]
    e = senders.shape[0]
    pad = EP - e

    x_up = _linear_up(node_feats, W_up0, W_up1)

    senders_p = jnp.concatenate([senders.astype(jnp.int32),
                                 jnp.zeros((pad,), jnp.int32)])
    receivers_p = jnp.concatenate([receivers.astype(jnp.int32),
                                   jnp.zeros((pad,), jnp.int32)])
    ev_p = jnp.concatenate([edge_vec, jnp.zeros((pad, 3), jnp.float32)])
    el_p = jnp.concatenate([edge_length[:, None],
                            jnp.zeros((pad, 1), jnp.float32)])

    x_up_i32 = lax.bitcast_convert_type(x_up.reshape(n, 2 * MUL, 2), jnp.int32)
    xs_i32 = _sc_gather(x_up_i32, senders_p.reshape(32, GATHER_CHUNKS, 128))
    xs = lax.bitcast_convert_type(xs_i32, jnp.bfloat16).reshape(EP, 4 * MUL)

    avg = e / jnp.maximum(jnp.asarray(num_nodes, jnp.float32), 1.0)
    scale = (1.0 / (np.sqrt(2.0 * MUL) * jnp.maximum(avg, 1e-8)))
    scale = scale.astype(jnp.float32).reshape(1, 1)

    ya, yb = _edge_compute(ev_p, el_p, xs, W1, b1, W2, b2, W3, b3, W_l0, W_l1,
                           scale, e)

    n_pad = 10240
    zeros_half = jnp.zeros((n_pad, 128), jnp.float32)
    acc = _sc_scatter(ya, yb, receivers_p.reshape(16, SCATTER_CHUNKS, 128),
                      zeros_half, n_pad)

    return jnp.concatenate([acc[0, :n], acc[1, :n]], axis=1)


# speedup vs baseline: 1.7144x; 1.7144x over previous
import functools

import jax
import jax.numpy as jnp
import numpy as np
from jax import lax
from jax.experimental import pallas as pl
from jax.experimental.pallas import tpu as pltpu
from jax.experimental.pallas import tpu_sc as plsc

MUL = 64
NB = 8
RMAX = 5.0
EP = 163840
GATHER_CHUNKS = 40
SCATTER_CHUNKS = 80


def _up_body(nf_ref, wu0_ref, wu1_ref, out_ref):
    nf = nf_ref[...]
    wu0 = wu0_ref[...]
    wu1 = wu1_ref[...]
    parts = [jnp.dot(nf[:, :MUL], wu0, preferred_element_type=jnp.float32)]
    for i in range(3):
        parts.append(jnp.dot(nf[:, MUL + 64 * i:MUL + 64 * (i + 1)], wu1,
                             preferred_element_type=jnp.float32))
    out_ref[...] = jnp.concatenate(parts, axis=1) * (1.0 / np.sqrt(MUL))


def _linear_up(node_feats, W_up0, W_up1):
    n = node_feats.shape[0]
    bn = 2000
    grid = n // bn
    return pl.pallas_call(
        _up_body,
        grid=(grid,),
        in_specs=[
            pl.BlockSpec((bn, 4 * MUL), lambda i: (i, 0)),
            pl.BlockSpec((MUL, MUL), lambda i: (0, 0)),
            pl.BlockSpec((MUL, MUL), lambda i: (0, 0)),
        ],
        out_specs=pl.BlockSpec((bn, 4 * MUL), lambda i: (i, 0)),
        out_shape=jax.ShapeDtypeStruct((n, 4 * MUL), jnp.float32),
    )(node_feats, W_up0, W_up1)


def _edge_body(ev_ref, el_ref, xs_ref, w1_ref, b1_ref, w2_ref, b2_ref,
               w3_ref, b3_ref, wl0_ref, wl1_ref, scale_ref, outa_ref, outb_ref,
               *, block_e, e_valid):
    ev = ev_ref[...]
    n2 = jnp.sum(ev * ev, axis=1, keepdims=True)
    u = ev / jnp.clip(jnp.sqrt(n2), 1e-12, None)
    sh = np.sqrt(3.0).astype(np.float32) * u

    r = el_ref[...]
    x = jnp.clip(r * (1.0 / RMAX), 0.0, 1.0)
    centers = (lax.broadcasted_iota(jnp.int32, (1, NB), 1).astype(jnp.float32)
               * (1.0 / (NB - 1)))
    d = (x - centers) * float(NB - 1)
    emb = jnp.exp(-0.5 * d * d) * np.sqrt(float(NB)).astype(np.float32)
    emb = emb * (r <= RMAX).astype(jnp.float32)

    h = emb @ w1_ref[...] + b1_ref[...]
    h = h * jax.nn.sigmoid(h)
    h = h @ w2_ref[...] + b2_ref[...]
    h = h * jax.nn.sigmoid(h)
    w = h @ w3_ref[...] + b3_ref[...]

    xi = xs_ref[...]
    xs_lo = lax.bitcast_convert_type(xi << 16, jnp.float32)
    xs_hi = lax.bitcast_convert_type(xi & jnp.int32(-65536), jnp.float32)
    x0s = xs_lo[:, :MUL]
    x1s = [xs_lo[:, MUL:], xs_hi[:, :MUL], xs_hi[:, MUL:]]
    dot31 = (x1s[0] * sh[:, 0:1] + x1s[1] * sh[:, 1:2] + x1s[2] * sh[:, 2:3])

    w0 = w[:, 0 * MUL:1 * MUL]
    w1 = w[:, 1 * MUL:2 * MUL]
    w2 = w[:, 2 * MUL:3 * MUL]
    w3 = w[:, 3 * MUL:4 * MUL]

    wl0 = wl0_ref[...]
    wl1 = wl1_ref[...]
    mid0 = jnp.concatenate([w0 * x0s, w3 * dot31 * (1.0 / np.sqrt(3.0))], axis=1)
    ys = [jnp.dot(mid0, wl0, preferred_element_type=jnp.float32)]
    w1x0 = w1 * x0s
    for i in range(3):
        mid1 = jnp.concatenate([w1x0 * sh[:, i:i + 1], w2 * x1s[i]], axis=1)
        ys.append(jnp.dot(mid1, wl1, preferred_element_type=jnp.float32))
    sc = scale_ref[0, 0]
    row = (pl.program_id(0) * block_e +
           lax.broadcasted_iota(jnp.int32, (ys[0].shape[0], 2 * MUL), 0))
    mask = (row < e_valid).astype(jnp.float32) * sc
    outa_ref[...] = jnp.concatenate(ys[:2], axis=1) * mask
    outb_ref[...] = jnp.concatenate(ys[2:], axis=1) * mask


def _edge_compute(ev_p, el_p, xs, W1, b1, W2, b2, W3, b3, W_l0, W_l1, scale,
                  e_valid):
    be = 2048
    grid = EP // be
    body = functools.partial(_edge_body, block_e=be, e_valid=e_valid)
    return pl.pallas_call(
        body,
        grid=(grid,),
        in_specs=[
            pl.BlockSpec((be, 3), lambda i: (i, 0)),
            pl.BlockSpec((be, 1), lambda i: (i, 0)),
            pl.BlockSpec((be, 2 * MUL), lambda i: (i, 0)),
            pl.BlockSpec((NB, 64), lambda i: (0, 0)),
            pl.BlockSpec((1, 64), lambda i: (0, 0)),
            pl.BlockSpec((64, 64), lambda i: (0, 0)),
            pl.BlockSpec((1, 64), lambda i: (0, 0)),
            pl.BlockSpec((64, 4 * MUL), lambda i: (0, 0)),
            pl.BlockSpec((1, 4 * MUL), lambda i: (0, 0)),
            pl.BlockSpec((2 * MUL, MUL), lambda i: (0, 0)),
            pl.BlockSpec((2 * MUL, MUL), lambda i: (0, 0)),
            pl.BlockSpec((1, 1), lambda i: (0, 0)),
        ],
        out_specs=[pl.BlockSpec((be, 2 * MUL), lambda i: (i, 0)),
                   pl.BlockSpec((be, 2 * MUL), lambda i: (i, 0))],
        out_shape=[jax.ShapeDtypeStruct((EP, 2 * MUL), jnp.float32),
                   jax.ShapeDtypeStruct((EP, 2 * MUL), jnp.float32)],
    )(ev_p, el_p, xs, W1, b1[None, :], W2, b2[None, :], W3, b3[None, :],
      W_l0, W_l1, scale)


def _sc_gather(x_up, senders3):
    mesh = plsc.VectorSubcoreMesh(core_axis_name="c", subcore_axis_name="s")
    d = 4 * MUL

    @functools.partial(
        pl.kernel, mesh=mesh,
        out_type=jax.ShapeDtypeStruct((EP, d // 2), jnp.int32),
        scratch_types=[
            pltpu.VMEM((GATHER_CHUNKS, 128), jnp.int32),
            pltpu.VMEM((128, d // 2), jnp.int32),
            pltpu.VMEM((128, d // 2), jnp.int32),
            pltpu.SemaphoreType.DMA,
            pltpu.SemaphoreType.DMA,
        ],
    )
    def gk(table_hbm, idx_hbm, out_hbm, idx_v, rows0_v, rows1_v, sem0, sem1):
        c = lax.axis_index("c")
        s = lax.axis_index("s")
        wid = s * 2 + c
        pltpu.sync_copy(idx_hbm.at[wid], idx_v)
        base = wid * (GATHER_CHUNKS * 128)

        def body(t, carry):
            j0 = 2 * t
            cp0 = pltpu.async_copy(table_hbm.at[idx_v.at[j0]], rows0_v, sem0)
            cp1 = pltpu.async_copy(table_hbm.at[idx_v.at[j0 + 1]], rows1_v, sem1)
            cp0.wait()
            pltpu.sync_copy(rows0_v, out_hbm.at[pl.ds(base + j0 * 128, 128)])
            cp1.wait()
            pltpu.sync_copy(rows1_v, out_hbm.at[pl.ds(base + j0 * 128 + 128, 128)])
            return carry

        lax.fori_loop(0, GATHER_CHUNKS // 2, body, 0)

    return gk(x_up, senders3)


def _sc_scatter(ya, yb, recv3, zeros_half, n_pad):
    mesh = plsc.VectorSubcoreMesh(core_axis_name="c", subcore_axis_name="s")
    rows_per_tile = n_pad // 16

    @functools.partial(
        pl.kernel, mesh=mesh,
        out_type=jax.ShapeDtypeStruct((2, n_pad, 128), jnp.float32),
        scratch_types=[
            pltpu.VMEM_SHARED((n_pad, 128), jnp.float32),
            pltpu.VMEM((SCATTER_CHUNKS, 128), jnp.int32),
            pltpu.VMEM((128, 128), jnp.float32),
        ],
    )
    def sk(ya_hbm, yb_hbm, idx_hbm, zeros_hbm, out_hbm, accum, idx_v, rows_v):
        c = lax.axis_index("c")
        s = lax.axis_index("s")
        nslice = pl.ds(s * rows_per_tile, rows_per_tile)
        pltpu.sync_copy(zeros_hbm.at[nslice], accum.at[nslice])
        pltpu.sync_copy(idx_hbm.at[s], idx_v)
        plsc.subcore_barrier()
        base = s * (SCATTER_CHUNKS * 128)

        def body(j, carry):
            sl = pl.ds(base + j * 128, 128)

            @pl.when(c == 0)
            def _():
                pltpu.sync_copy(ya_hbm.at[sl], rows_v)

            @pl.when(c == 1)
            def _():
                pltpu.sync_copy(yb_hbm.at[sl], rows_v)

            pltpu.sync_copy(rows_v, accum.at[idx_v.at[j]], add=True)
            return carry

        lax.fori_loop(0, SCATTER_CHUNKS, body, 0)
        plsc.subcore_barrier()
        pltpu.sync_copy(accum.at[nslice], out_hbm.at[c].at[nslice])

    return sk(ya, yb, recv3, zeros_half)


def kernel(node_feats, edge_vec, edge_length, senders, receivers, num_nodes,
           W1, b1, W2, b2, W3, b3, W_up0, W_up1, W_l0, W_l1):
    n = node_feats.shape[0]
    e = senders.shape[0]
    pad = EP - e

    x_up = _linear_up(node_feats, W_up0, W_up1)

    senders_p = jnp.concatenate([senders.astype(jnp.int32),
                                 jnp.zeros((pad,), jnp.int32)])
    receivers_p = jnp.concatenate([receivers.astype(jnp.int32),
                                   jnp.zeros((pad,), jnp.int32)])
    ev_p = jnp.concatenate([edge_vec, jnp.zeros((pad, 3), jnp.float32)])
    el_p = jnp.concatenate([edge_length[:, None],
                            jnp.zeros((pad, 1), jnp.float32)])

    xa = lax.bitcast_convert_type(x_up[:, :2 * MUL].astype(jnp.bfloat16),
                                  jnp.uint16).astype(jnp.uint32)
    xb = lax.bitcast_convert_type(x_up[:, 2 * MUL:].astype(jnp.bfloat16),
                                  jnp.uint16).astype(jnp.uint32)
    x_up_i32 = lax.bitcast_convert_type(xa | (xb << 16), jnp.int32)
    xs = _sc_gather(x_up_i32, senders_p.reshape(32, GATHER_CHUNKS, 128))

    avg = e / jnp.maximum(jnp.asarray(num_nodes, jnp.float32), 1.0)
    scale = (1.0 / (np.sqrt(2.0 * MUL) * jnp.maximum(avg, 1e-8)))
    scale = scale.astype(jnp.float32).reshape(1, 1)

    ya, yb = _edge_compute(ev_p, el_p, xs, W1, b1, W2, b2, W3, b3, W_l0, W_l1,
                           scale, e)

    n_pad = 10240
    zeros_half = jnp.zeros((n_pad, 128), jnp.float32)
    acc = _sc_scatter(ya, yb, receivers_p.reshape(16, SCATTER_CHUNKS, 128),
                      zeros_half, n_pad)

    return jnp.concatenate([acc[0, :n], acc[1, :n]], axis=1)


# speedup vs baseline: 1.7611x; 1.0272x over previous
import functools

import jax
import jax.numpy as jnp
import numpy as np
from jax import lax
from jax.experimental import pallas as pl
from jax.experimental.pallas import tpu as pltpu
from jax.experimental.pallas import tpu_sc as plsc

MUL = 64
NB = 8
RMAX = 5.0
EP = 163840
GATHER_CHUNKS = 40
SCATTER_CHUNKS = 80


def _up_body(nf_ref, wu0_ref, wu1_ref, out_ref):
    nf = nf_ref[...]
    wu0 = wu0_ref[...]
    wu1 = wu1_ref[...]
    parts = [jnp.dot(nf[:, :MUL], wu0, preferred_element_type=jnp.float32)]
    for i in range(3):
        parts.append(jnp.dot(nf[:, MUL + 64 * i:MUL + 64 * (i + 1)], wu1,
                             preferred_element_type=jnp.float32))
    x = jnp.concatenate(parts, axis=1) * (1.0 / np.sqrt(MUL))
    lo = lax.bitcast_convert_type(x[:, :2 * MUL].astype(jnp.bfloat16),
                                  jnp.uint16).astype(jnp.uint32)
    hi = lax.bitcast_convert_type(x[:, 2 * MUL:].astype(jnp.bfloat16),
                                  jnp.uint16).astype(jnp.uint32)
    out_ref[...] = lax.bitcast_convert_type(lo | (hi << 16), jnp.int32)


def _linear_up(node_feats, W_up0, W_up1):
    n = node_feats.shape[0]
    bn = 2000
    grid = n // bn
    return pl.pallas_call(
        _up_body,
        grid=(grid,),
        in_specs=[
            pl.BlockSpec((bn, 4 * MUL), lambda i: (i, 0)),
            pl.BlockSpec((MUL, MUL), lambda i: (0, 0)),
            pl.BlockSpec((MUL, MUL), lambda i: (0, 0)),
        ],
        out_specs=pl.BlockSpec((bn, 2 * MUL), lambda i: (i, 0)),
        out_shape=jax.ShapeDtypeStruct((n, 2 * MUL), jnp.int32),
    )(node_feats, W_up0, W_up1)


def _edge_body(ev_ref, el_ref, xs_ref, w1_ref, b1_ref, w2_ref, b2_ref,
               w3_ref, b3_ref, wl0_ref, wl1_ref, scale_ref, outa_ref, outb_ref,
               *, block_e, e_valid):
    ev = ev_ref[...]
    n2 = jnp.sum(ev * ev, axis=1, keepdims=True)
    u = ev / jnp.clip(jnp.sqrt(n2), 1e-12, None)
    sh = np.sqrt(3.0).astype(np.float32) * u

    r = el_ref[...]
    x = jnp.clip(r * (1.0 / RMAX), 0.0, 1.0)
    centers = (lax.broadcasted_iota(jnp.int32, (1, NB), 1).astype(jnp.float32)
               * (1.0 / (NB - 1)))
    d = (x - centers) * float(NB - 1)
    emb = jnp.exp(-0.5 * d * d) * np.sqrt(float(NB)).astype(np.float32)
    emb = emb * (r <= RMAX).astype(jnp.float32)

    h = emb @ w1_ref[...] + b1_ref[...]
    h = h * jax.nn.sigmoid(h)
    h = h @ w2_ref[...] + b2_ref[...]
    h = h * jax.nn.sigmoid(h)
    w = h @ w3_ref[...] + b3_ref[...]

    xi = xs_ref[...]
    xs_lo = lax.bitcast_convert_type(xi << 16, jnp.float32)
    xs_hi = lax.bitcast_convert_type(xi & jnp.int32(-65536), jnp.float32)
    x0s = xs_lo[:, :MUL]
    x1s = [xs_lo[:, MUL:], xs_hi[:, :MUL], xs_hi[:, MUL:]]
    dot31 = (x1s[0] * sh[:, 0:1] + x1s[1] * sh[:, 1:2] + x1s[2] * sh[:, 2:3])

    w0 = w[:, 0 * MUL:1 * MUL]
    w1 = w[:, 1 * MUL:2 * MUL]
    w2 = w[:, 2 * MUL:3 * MUL]
    w3 = w[:, 3 * MUL:4 * MUL]

    wl0 = wl0_ref[...]
    wl1 = wl1_ref[...]
    mid0 = jnp.concatenate([w0 * x0s, w3 * dot31 * (1.0 / np.sqrt(3.0))], axis=1)
    ys = [jnp.dot(mid0, wl0, preferred_element_type=jnp.float32)]
    w1x0 = w1 * x0s
    for i in range(3):
        mid1 = jnp.concatenate([w1x0 * sh[:, i:i + 1], w2 * x1s[i]], axis=1)
        ys.append(jnp.dot(mid1, wl1, preferred_element_type=jnp.float32))
    sc = scale_ref[0, 0]
    row = (pl.program_id(0) * block_e +
           lax.broadcasted_iota(jnp.int32, (ys[0].shape[0], 2 * MUL), 0))
    mask = (row < e_valid).astype(jnp.float32) * sc
    outa_ref[...] = jnp.concatenate(ys[:2], axis=1) * mask
    outb_ref[...] = jnp.concatenate(ys[2:], axis=1) * mask


def _edge_compute(ev_p, el_p, xs, W1, b1, W2, b2, W3, b3, W_l0, W_l1, scale,
                  e_valid):
    be = 2048
    grid = EP // be
    body = functools.partial(_edge_body, block_e=be, e_valid=e_valid)
    return pl.pallas_call(
        body,
        grid=(grid,),
        in_specs=[
            pl.BlockSpec((be, 3), lambda i: (i, 0)),
            pl.BlockSpec((be, 1), lambda i: (i, 0)),
            pl.BlockSpec((be, 2 * MUL), lambda i: (i, 0)),
            pl.BlockSpec((NB, 64), lambda i: (0, 0)),
            pl.BlockSpec((1, 64), lambda i: (0, 0)),
            pl.BlockSpec((64, 64), lambda i: (0, 0)),
            pl.BlockSpec((1, 64), lambda i: (0, 0)),
            pl.BlockSpec((64, 4 * MUL), lambda i: (0, 0)),
            pl.BlockSpec((1, 4 * MUL), lambda i: (0, 0)),
            pl.BlockSpec((2 * MUL, MUL), lambda i: (0, 0)),
            pl.BlockSpec((2 * MUL, MUL), lambda i: (0, 0)),
            pl.BlockSpec((1, 1), lambda i: (0, 0)),
        ],
        out_specs=[pl.BlockSpec((be, 2 * MUL), lambda i: (i, 0)),
                   pl.BlockSpec((be, 2 * MUL), lambda i: (i, 0))],
        out_shape=[jax.ShapeDtypeStruct((EP, 2 * MUL), jnp.float32),
                   jax.ShapeDtypeStruct((EP, 2 * MUL), jnp.float32)],
    )(ev_p, el_p, xs, W1, b1[None, :], W2, b2[None, :], W3, b3[None, :],
      W_l0, W_l1, scale)


def _sc_gather(x_up, senders3):
    mesh = plsc.VectorSubcoreMesh(core_axis_name="c", subcore_axis_name="s")
    d = 4 * MUL

    nbuf = 4
    chunk = 64
    nchunks = EP // (32 * chunk)

    @functools.partial(
        pl.kernel, mesh=mesh,
        out_type=jax.ShapeDtypeStruct((EP, d // 2), jnp.int32),
        scratch_types=[
            pltpu.VMEM((nchunks, chunk), jnp.int32),
        ] + [pltpu.VMEM((chunk, d // 2), jnp.int32)] * nbuf
          + [pltpu.SemaphoreType.DMA] * nbuf,
    )
    def gk(table_hbm, idx_hbm, out_hbm, idx_v, *bufs_and_sems):
        bufs = bufs_and_sems[:nbuf]
        sems = bufs_and_sems[nbuf:]
        c = lax.axis_index("c")
        s = lax.axis_index("s")
        wid = s * 2 + c
        pltpu.sync_copy(idx_hbm.at[wid], idx_v)
        base = wid * (nchunks * chunk)

        def body(t, carry):
            j0 = nbuf * t
            cps = [pltpu.async_copy(table_hbm.at[idx_v.at[j0 + b]], bufs[b],
                                    sems[b]) for b in range(nbuf)]
            for b in range(nbuf):
                cps[b].wait()
                pltpu.sync_copy(
                    bufs[b], out_hbm.at[pl.ds(base + (j0 + b) * chunk, chunk)])
            return carry

        lax.fori_loop(0, nchunks // nbuf, body, 0)

    return gk(x_up, senders3)


def _sc_scatter(ya, yb, recv3, zeros_half, n_pad):
    mesh = plsc.VectorSubcoreMesh(core_axis_name="c", subcore_axis_name="s")
    rows_per_tile = n_pad // 16

    @functools.partial(
        pl.kernel, mesh=mesh,
        out_type=jax.ShapeDtypeStruct((2, n_pad, 128), jnp.float32),
        scratch_types=[
            pltpu.VMEM_SHARED((n_pad, 128), jnp.float32),
            pltpu.VMEM((SCATTER_CHUNKS, 128), jnp.int32),
            pltpu.VMEM((128, 128), jnp.float32),
            pltpu.VMEM((128, 128), jnp.float32),
            pltpu.SemaphoreType.DMA,
            pltpu.SemaphoreType.DMA,
        ],
    )
    def sk(ya_hbm, yb_hbm, idx_hbm, zeros_hbm, out_hbm, accum, idx_v,
           rows0_v, rows1_v, sem0, sem1):
        c = lax.axis_index("c")
        s = lax.axis_index("s")
        nslice = pl.ds(s * rows_per_tile, rows_per_tile)
        pltpu.sync_copy(zeros_hbm.at[nslice], accum.at[nslice])
        pltpu.sync_copy(idx_hbm.at[s], idx_v)
        plsc.subcore_barrier()
        base = s * (SCATTER_CHUNKS * 128)

        def body(t, carry):
            j0 = 2 * t
            sl0 = pl.ds(base + j0 * 128, 128)
            sl1 = pl.ds(base + j0 * 128 + 128, 128)

            @pl.when(c == 0)
            def _():
                cp0 = pltpu.async_copy(ya_hbm.at[sl0], rows0_v, sem0)
                cp1 = pltpu.async_copy(ya_hbm.at[sl1], rows1_v, sem1)
                cp0.wait()
                pltpu.sync_copy(rows0_v, accum.at[idx_v.at[j0]], add=True)
                cp1.wait()
                pltpu.sync_copy(rows1_v, accum.at[idx_v.at[j0 + 1]], add=True)

            @pl.when(c == 1)
            def _():
                cp0 = pltpu.async_copy(yb_hbm.at[sl0], rows0_v, sem0)
                cp1 = pltpu.async_copy(yb_hbm.at[sl1], rows1_v, sem1)
                cp0.wait()
                pltpu.sync_copy(rows0_v, accum.at[idx_v.at[j0]], add=True)
                cp1.wait()
                pltpu.sync_copy(rows1_v, accum.at[idx_v.at[j0 + 1]], add=True)

            return carry

        lax.fori_loop(0, SCATTER_CHUNKS // 2, body, 0)
        plsc.subcore_barrier()
        pltpu.sync_copy(accum.at[nslice], out_hbm.at[c].at[nslice])

    return sk(ya, yb, recv3, zeros_half)


def kernel(node_feats, edge_vec, edge_length, senders, receivers, num_nodes,
           W1, b1, W2, b2, W3, b3, W_up0, W_up1, W_l0, W_l1):
    n = node_feats.shape[0]
    e = senders.shape[0]
    pad = EP - e

    x_up = _linear_up(node_feats, W_up0, W_up1)

    senders_p = jnp.concatenate([senders.astype(jnp.int32),
                                 jnp.zeros((pad,), jnp.int32)])
    receivers_p = jnp.concatenate([receivers.astype(jnp.int32),
                                   jnp.zeros((pad,), jnp.int32)])
    ev_p = jnp.concatenate([edge_vec, jnp.zeros((pad, 3), jnp.float32)])
    el_p = jnp.concatenate([edge_length[:, None],
                            jnp.zeros((pad, 1), jnp.float32)])

    xs = _sc_gather(x_up, senders_p.reshape(32, EP // (32 * 64), 64))

    avg = e / jnp.maximum(jnp.asarray(num_nodes, jnp.float32), 1.0)
    scale = (1.0 / (np.sqrt(2.0 * MUL) * jnp.maximum(avg, 1e-8)))
    scale = scale.astype(jnp.float32).reshape(1, 1)

    ya, yb = _edge_compute(ev_p, el_p, xs, W1, b1, W2, b2, W3, b3, W_l0, W_l1,
                           scale, e)

    n_pad = 10240
    zeros_half = jnp.zeros((n_pad, 128), jnp.float32)
    acc = _sc_scatter(ya, yb, receivers_p.reshape(16, SCATTER_CHUNKS, 128),
                      zeros_half, n_pad)

    return jnp.concatenate([acc[0, :n], acc[1, :n]], axis=1)
